# Initial kernel scaffold; baseline (speedup 1.0000x reference)
import jax
import jax.numpy as jnp
from jax.experimental import pallas as pl


def kernel(input_cloud, W1, b1, W2, b2):
    raise NotImplementedError("write your pallas kernel here")


# speedup vs baseline: 16.6726x; 16.6726x over previous
import functools

import jax
import jax.numpy as jnp
from jax.experimental import pallas as pl
from jax.experimental.pallas import tpu as pltpu

_K = 10


def _eigvals_sym3(a, b, c, d, e, f):
    q = (a + b + c) * (1.0 / 3.0)
    p1 = d * d + e * e + f * f
    aq = a - q
    bq = b - q
    cq = c - q
    p2 = aq * aq + bq * bq + cq * cq + 2.0 * p1
    p = jnp.sqrt(p2 * (1.0 / 6.0))
    ps = jnp.maximum(p, 1e-30)
    inv = 1.0 / ps
    b00 = aq * inv
    b11 = bq * inv
    b22 = cq * inv
    b01 = d * inv
    b02 = e * inv
    b12 = f * inv
    detb = (
        b00 * (b11 * b22 - b12 * b12)
        - b01 * (b01 * b22 - b12 * b02)
        + b02 * (b01 * b12 - b11 * b02)
    )
    r = jnp.clip(0.5 * detb, -1.0, 1.0)
    phi = jnp.arctan2(jnp.sqrt(jnp.maximum(1.0 - r * r, 0.0)), r) * (1.0 / 3.0)
    two_pi_3 = 2.0943951023931953
    l1 = q + 2.0 * p * jnp.cos(phi)
    l3 = q + 2.0 * p * jnp.cos(phi + two_pi_3)
    l2 = 3.0 * q - l1 - l3
    return l1, l2, l3


def _lge_kernel(q_ref, ptst_ref, pts_ref, w1t_ref, b1_ref, w2t_ref, b2_ref,
                out_ref, *, blk, n):
    f32 = jnp.float32
    qp = q_ref[0]
    ptst = ptst_ref[0]
    pts = pts_ref[0]

    qx = qp[:, 0:1]
    qy = qp[:, 1:2]
    qz = qp[:, 2:3]
    kx = ptst[0:1, :]
    ky = ptst[1:2, :]
    kz = ptst[2:3, :]

    q2 = qx * qx + qy * qy + qz * qz
    k2 = kx * kx + ky * ky + kz * kz
    dot = jax.lax.dot_general(
        qp.astype(jnp.bfloat16), ptst.astype(jnp.bfloat16),
        (((1,), (0,)), ((), ())),
        preferred_element_type=f32)
    d2 = (q2 + k2) - 2.0 * dot

    iota = jax.lax.broadcasted_iota(jnp.int32, (blk, n), 1)
    big = jnp.float32(jnp.inf)

    nbs = []
    for t in range(_K + 1):
        idx = jnp.argmin(d2, axis=1, keepdims=True)
        sel = iota == idx
        if t > 0:
            onehot = sel.astype(f32)
            nb = jax.lax.dot_general(
                onehot, pts, (((1,), (0,)), ((), ())),
                preferred_element_type=f32,
                precision=jax.lax.Precision.HIGHEST,
            )
            nbs.append(nb)
        d2 = jnp.where(sel, big, d2)

    nb_flat = jnp.concatenate(nbs, axis=1)
    rels = [nb - qp for nb in nbs]
    rel_flat = jnp.concatenate(rels, axis=1)
    dists = jnp.concatenate(
        [jnp.sqrt(jnp.sum(r * r, axis=1, keepdims=True)) for r in rels],
        axis=1)

    mean = nbs[0]
    for nb in nbs[1:]:
        mean = mean + nb
    mean = mean * (1.0 / _K)
    c00 = c11 = c22 = c01 = c02 = c12 = jnp.zeros((blk, 1), f32)
    for nb in nbs:
        cx = nb[:, 0:1] - mean[:, 0:1]
        cy = nb[:, 1:2] - mean[:, 1:2]
        cz = nb[:, 2:3] - mean[:, 2:3]
        c00 = c00 + cx * cx
        c11 = c11 + cy * cy
        c22 = c22 + cz * cz
        c01 = c01 + cx * cy
        c02 = c02 + cx * cz
        c12 = c12 + cy * cz
    s = 1.0 / (_K - 1)
    l1, l2, l3 = _eigvals_sym3(c00 * s, c11 * s, c22 * s,
                               c01 * s, c02 * s, c12 * s)
    inv_l1 = 1.0 / l1
    linearity = (l1 - l2) * inv_l1
    planarity = (l2 - l3) * inv_l1
    scattering = l3 * inv_l1

    feats = jnp.concatenate(
        [qp, nb_flat, rel_flat, dists, linearity, planarity, scattering],
        axis=1)

    h = jax.lax.dot_general(
        feats, w1t_ref[0], (((1,), (0,)), ((), ())),
        preferred_element_type=f32,
        precision=jax.lax.Precision.HIGHEST) + b1_ref[0]
    h = jnp.maximum(h, 0.0)
    o = jax.lax.dot_general(
        h, w2t_ref[0], (((1,), (0,)), ((), ())),
        preferred_element_type=f32,
        precision=jax.lax.Precision.HIGHEST) + b2_ref[0]
    out_ref[0] = jnp.maximum(o, 0.0)


@jax.jit
def kernel(input_cloud, W1, b1, W2, b2):
    B, N, _ = input_cloud.shape
    BLK = 256
    pts_t = jnp.transpose(input_cloud, (0, 2, 1))
    w1t = jnp.transpose(W1)[None]
    w2t = jnp.transpose(W2)[None]
    b1r = b1[None, None, :]
    b2r = b2[None, None, :]

    grid = (B, N // BLK)
    out = pl.pallas_call(
        functools.partial(_lge_kernel, blk=BLK, n=N),
        grid=grid,
        in_specs=[
            pl.BlockSpec((1, BLK, 3), lambda b, i: (b, i, 0)),
            pl.BlockSpec((1, 3, N), lambda b, i: (b, 0, 0)),
            pl.BlockSpec((1, N, 3), lambda b, i: (b, 0, 0)),
            pl.BlockSpec((1, 76, 64), lambda b, i: (0, 0, 0)),
            pl.BlockSpec((1, 1, 64), lambda b, i: (0, 0, 0)),
            pl.BlockSpec((1, 64, 3), lambda b, i: (0, 0, 0)),
            pl.BlockSpec((1, 1, 3), lambda b, i: (0, 0, 0)),
        ],
        out_specs=pl.BlockSpec((1, BLK, 3), lambda b, i: (b, i, 0)),
        out_shape=jax.ShapeDtypeStruct((B, N, 3), jnp.float32),
    )(input_cloud, pts_t, input_cloud, w1t, b1r, w2t, b2r)
    return out


# speedup vs baseline: 30.1719x; 1.8097x over previous
import functools

import jax
import jax.numpy as jnp
from jax.experimental import pallas as pl
from jax.experimental.pallas import tpu as pltpu

_K = 10


def _eigvals_sym3(a, b, c, d, e, f):
    q = (a + b + c) * (1.0 / 3.0)
    p1 = d * d + e * e + f * f
    aq = a - q
    bq = b - q
    cq = c - q
    p2 = aq * aq + bq * bq + cq * cq + 2.0 * p1
    p = jnp.sqrt(p2 * (1.0 / 6.0))
    ps = jnp.maximum(p, 1e-30)
    inv = 1.0 / ps
    b00 = aq * inv
    b11 = bq * inv
    b22 = cq * inv
    b01 = d * inv
    b02 = e * inv
    b12 = f * inv
    detb = (
        b00 * (b11 * b22 - b12 * b12)
        - b01 * (b01 * b22 - b12 * b02)
        + b02 * (b01 * b12 - b11 * b02)
    )
    r = jnp.clip(0.5 * detb, -1.0, 1.0)
    phi = jnp.arctan2(jnp.sqrt(jnp.maximum(1.0 - r * r, 0.0)), r) * (1.0 / 3.0)
    two_pi_3 = 2.0943951023931953
    l1 = q + 2.0 * p * jnp.cos(phi)
    l3 = q + 2.0 * p * jnp.cos(phi + two_pi_3)
    l2 = 3.0 * q - l1 - l3
    return l1, l2, l3


def _lge_kernel(q_ref, ptst_ref, w1t_ref, b1_ref, w2t_ref, b2_ref,
                out_ref, *, blk, n):
    f32 = jnp.float32
    qp = q_ref[0]
    ptst = ptst_ref[0]

    qx = qp[:, 0:1]
    qy = qp[:, 1:2]
    qz = qp[:, 2:3]
    kx = ptst[0:1, :]
    ky = ptst[1:2, :]
    kz = ptst[2:3, :]

    q2 = qx * qx + qy * qy + qz * qz
    k2 = kx * kx + ky * ky + kz * kz
    dot = jax.lax.dot_general(
        qp.astype(jnp.bfloat16), ptst.astype(jnp.bfloat16),
        (((1,), (0,)), ((), ())),
        preferred_element_type=f32)
    d2 = (q2 + k2) - 2.0 * dot

    iota = jax.lax.broadcasted_iota(jnp.int32, (blk, n), 1)
    big = jnp.float32(jnp.inf)

    nbs = []
    for t in range(_K + 1):
        idx = jnp.argmin(d2, axis=1, keepdims=True)
        sel = iota == idx
        if t > 0:
            zero = jnp.float32(0.0)
            nbx = jnp.sum(jnp.where(sel, kx, zero), axis=1, keepdims=True)
            nby = jnp.sum(jnp.where(sel, ky, zero), axis=1, keepdims=True)
            nbz = jnp.sum(jnp.where(sel, kz, zero), axis=1, keepdims=True)
            nbs.append(jnp.concatenate([nbx, nby, nbz], axis=1))
        d2 = jnp.where(sel, big, d2)

    nb_flat = jnp.concatenate(nbs, axis=1)
    rels = [nb - qp for nb in nbs]
    rel_flat = jnp.concatenate(rels, axis=1)
    dists = jnp.concatenate(
        [jnp.sqrt(jnp.sum(r * r, axis=1, keepdims=True)) for r in rels],
        axis=1)

    mean = nbs[0]
    for nb in nbs[1:]:
        mean = mean + nb
    mean = mean * (1.0 / _K)
    c00 = c11 = c22 = c01 = c02 = c12 = jnp.zeros((blk, 1), f32)
    for nb in nbs:
        cx = nb[:, 0:1] - mean[:, 0:1]
        cy = nb[:, 1:2] - mean[:, 1:2]
        cz = nb[:, 2:3] - mean[:, 2:3]
        c00 = c00 + cx * cx
        c11 = c11 + cy * cy
        c22 = c22 + cz * cz
        c01 = c01 + cx * cy
        c02 = c02 + cx * cz
        c12 = c12 + cy * cz
    s = 1.0 / (_K - 1)
    l1, l2, l3 = _eigvals_sym3(c00 * s, c11 * s, c22 * s,
                               c01 * s, c02 * s, c12 * s)
    inv_l1 = 1.0 / l1
    linearity = (l1 - l2) * inv_l1
    planarity = (l2 - l3) * inv_l1
    scattering = l3 * inv_l1

    feats = jnp.concatenate(
        [qp, nb_flat, rel_flat, dists, linearity, planarity, scattering],
        axis=1)

    h = jax.lax.dot_general(
        feats, w1t_ref[0], (((1,), (0,)), ((), ())),
        preferred_element_type=f32,
        precision=jax.lax.Precision.HIGHEST) + b1_ref[0]
    h = jnp.maximum(h, 0.0)
    o = jax.lax.dot_general(
        h, w2t_ref[0], (((1,), (0,)), ((), ())),
        preferred_element_type=f32,
        precision=jax.lax.Precision.HIGHEST) + b2_ref[0]
    out_ref[0] = jnp.maximum(o, 0.0)


@jax.jit
def kernel(input_cloud, W1, b1, W2, b2):
    B, N, _ = input_cloud.shape
    BLK = 512
    pts_t = jnp.transpose(input_cloud, (0, 2, 1))
    w1t = jnp.transpose(W1)[None]
    w2t = jnp.transpose(W2)[None]
    b1r = b1[None, None, :]
    b2r = b2[None, None, :]

    grid = (B, N // BLK)
    out = pl.pallas_call(
        functools.partial(_lge_kernel, blk=BLK, n=N),
        grid=grid,
        in_specs=[
            pl.BlockSpec((1, BLK, 3), lambda b, i: (b, i, 0)),
            pl.BlockSpec((1, 3, N), lambda b, i: (b, 0, 0)),
            pl.BlockSpec((1, 76, 64), lambda b, i: (0, 0, 0)),
            pl.BlockSpec((1, 1, 64), lambda b, i: (0, 0, 0)),
            pl.BlockSpec((1, 64, 3), lambda b, i: (0, 0, 0)),
            pl.BlockSpec((1, 1, 3), lambda b, i: (0, 0, 0)),
        ],
        out_specs=pl.BlockSpec((1, BLK, 3), lambda b, i: (b, i, 0)),
        out_shape=jax.ShapeDtypeStruct((B, N, 3), jnp.float32),
    )(input_cloud, pts_t, w1t, b1r, w2t, b2r)
    return out


# speedup vs baseline: 30.6794x; 1.0168x over previous
import functools

import jax
import jax.numpy as jnp
from jax.experimental import pallas as pl
from jax.experimental.pallas import tpu as pltpu

_K = 10


def _eigvals_sym3(a, b, c, d, e, f):
    q = (a + b + c) * (1.0 / 3.0)
    p1 = d * d + e * e + f * f
    aq = a - q
    bq = b - q
    cq = c - q
    p2 = aq * aq + bq * bq + cq * cq + 2.0 * p1
    p = jnp.sqrt(p2 * (1.0 / 6.0))
    ps = jnp.maximum(p, 1e-30)
    inv = 1.0 / ps
    b00 = aq * inv
    b11 = bq * inv
    b22 = cq * inv
    b01 = d * inv
    b02 = e * inv
    b12 = f * inv
    detb = (
        b00 * (b11 * b22 - b12 * b12)
        - b01 * (b01 * b22 - b12 * b02)
        + b02 * (b01 * b12 - b11 * b02)
    )
    r = jnp.clip(0.5 * detb, -1.0, 1.0)
    phi = jnp.arctan2(jnp.sqrt(jnp.maximum(1.0 - r * r, 0.0)), r) * (1.0 / 3.0)
    two_pi_3 = 2.0943951023931953
    l1 = q + 2.0 * p * jnp.cos(phi)
    l3 = q + 2.0 * p * jnp.cos(phi + two_pi_3)
    l2 = 3.0 * q - l1 - l3
    return l1, l2, l3


def _lge_kernel(q_ref, ptst_ref, w1t_ref, b1_ref, w2t_ref, b2_ref,
                out_ref, *, blk, n):
    f32 = jnp.float32
    qp = q_ref[0]
    ptst = ptst_ref[0]

    qx = qp[:, 0:1]
    qy = qp[:, 1:2]
    qz = qp[:, 2:3]
    kx = ptst[0:1, :]
    ky = ptst[1:2, :]
    kz = ptst[2:3, :]

    q2 = qx * qx + qy * qy + qz * qz
    k2 = kx * kx + ky * ky + kz * kz
    dot = jax.lax.dot_general(
        qp.astype(jnp.bfloat16), ptst.astype(jnp.bfloat16),
        (((1,), (0,)), ((), ())),
        preferred_element_type=f32)
    d2 = (q2 + k2) - 2.0 * dot

    iota = jax.lax.broadcasted_iota(jnp.int32, (blk, n), 1)
    big = jnp.float32(jnp.inf)

    nbs = []
    for t in range(_K + 1):
        idx = jnp.argmin(d2, axis=1, keepdims=True)
        sel = iota == idx
        if t > 0:
            zero = jnp.float32(0.0)
            nbx = jnp.sum(jnp.where(sel, kx, zero), axis=1, keepdims=True)
            nby = jnp.sum(jnp.where(sel, ky, zero), axis=1, keepdims=True)
            nbz = jnp.sum(jnp.where(sel, kz, zero), axis=1, keepdims=True)
            nbs.append(jnp.concatenate([nbx, nby, nbz], axis=1))
        d2 = jnp.where(sel, big, d2)

    nb_flat = jnp.concatenate(nbs, axis=1)
    rels = [nb - qp for nb in nbs]
    rel_flat = jnp.concatenate(rels, axis=1)
    dists = jnp.concatenate(
        [jnp.sqrt(jnp.sum(r * r, axis=1, keepdims=True)) for r in rels],
        axis=1)

    mean = nbs[0]
    for nb in nbs[1:]:
        mean = mean + nb
    mean = mean * (1.0 / _K)
    c00 = c11 = c22 = c01 = c02 = c12 = jnp.zeros((blk, 1), f32)
    for nb in nbs:
        cx = nb[:, 0:1] - mean[:, 0:1]
        cy = nb[:, 1:2] - mean[:, 1:2]
        cz = nb[:, 2:3] - mean[:, 2:3]
        c00 = c00 + cx * cx
        c11 = c11 + cy * cy
        c22 = c22 + cz * cz
        c01 = c01 + cx * cy
        c02 = c02 + cx * cz
        c12 = c12 + cy * cz
    s = 1.0 / (_K - 1)
    l1, l2, l3 = _eigvals_sym3(c00 * s, c11 * s, c22 * s,
                               c01 * s, c02 * s, c12 * s)
    inv_l1 = 1.0 / l1
    linearity = (l1 - l2) * inv_l1
    planarity = (l2 - l3) * inv_l1
    scattering = l3 * inv_l1

    feats = jnp.concatenate(
        [qp, nb_flat, rel_flat, dists, linearity, planarity, scattering],
        axis=1)

    h = jax.lax.dot_general(
        feats.astype(jnp.bfloat16), w1t_ref[0].astype(jnp.bfloat16),
        (((1,), (0,)), ((), ())),
        preferred_element_type=f32) + b1_ref[0]
    h = jnp.maximum(h, 0.0)
    o = jax.lax.dot_general(
        h.astype(jnp.bfloat16), w2t_ref[0].astype(jnp.bfloat16),
        (((1,), (0,)), ((), ())),
        preferred_element_type=f32) + b2_ref[0]
    out_ref[0] = jnp.maximum(o, 0.0)


@jax.jit
def kernel(input_cloud, W1, b1, W2, b2):
    B, N, _ = input_cloud.shape
    BLK = 512
    pts_t = jnp.transpose(input_cloud, (0, 2, 1))
    w1t = jnp.transpose(W1)[None]
    w2t = jnp.transpose(W2)[None]
    b1r = b1[None, None, :]
    b2r = b2[None, None, :]

    grid = (B, N // BLK)
    out = pl.pallas_call(
        functools.partial(_lge_kernel, blk=BLK, n=N),
        grid=grid,
        in_specs=[
            pl.BlockSpec((1, BLK, 3), lambda b, i: (b, i, 0)),
            pl.BlockSpec((1, 3, N), lambda b, i: (b, 0, 0)),
            pl.BlockSpec((1, 76, 64), lambda b, i: (0, 0, 0)),
            pl.BlockSpec((1, 1, 64), lambda b, i: (0, 0, 0)),
            pl.BlockSpec((1, 64, 3), lambda b, i: (0, 0, 0)),
            pl.BlockSpec((1, 1, 3), lambda b, i: (0, 0, 0)),
        ],
        out_specs=pl.BlockSpec((1, BLK, 3), lambda b, i: (b, i, 0)),
        out_shape=jax.ShapeDtypeStruct((B, N, 3), jnp.float32),
        compiler_params=pltpu.CompilerParams(
            dimension_semantics=("parallel", "parallel")),
    )(input_cloud, pts_t, w1t, b1r, w2t, b2r)
    return out


# speedup vs baseline: 33.6887x; 1.0981x over previous
import functools

import jax
import jax.numpy as jnp
from jax.experimental import pallas as pl
from jax.experimental.pallas import tpu as pltpu

_K = 10


def _eigvals_sym3(a, b, c, d, e, f):
    q = (a + b + c) * (1.0 / 3.0)
    p1 = d * d + e * e + f * f
    aq = a - q
    bq = b - q
    cq = c - q
    p2 = aq * aq + bq * bq + cq * cq + 2.0 * p1
    p = jnp.sqrt(p2 * (1.0 / 6.0))
    ps = jnp.maximum(p, 1e-30)
    inv = 1.0 / ps
    b00 = aq * inv
    b11 = bq * inv
    b22 = cq * inv
    b01 = d * inv
    b02 = e * inv
    b12 = f * inv
    detb = (
        b00 * (b11 * b22 - b12 * b12)
        - b01 * (b01 * b22 - b12 * b02)
        + b02 * (b01 * b12 - b11 * b02)
    )
    r = jnp.clip(0.5 * detb, -1.0, 1.0)
    phi = jnp.arctan2(jnp.sqrt(jnp.maximum(1.0 - r * r, 0.0)), r) * (1.0 / 3.0)
    two_pi_3 = 2.0943951023931953
    l1 = q + 2.0 * p * jnp.cos(phi)
    l3 = q + 2.0 * p * jnp.cos(phi + two_pi_3)
    l2 = 3.0 * q - l1 - l3
    return l1, l2, l3


def _lge_kernel(q_ref, ptst_ref, w1t_ref, b1_ref, w2t_ref, b2_ref,
                out_ref, *, blk, n):
    f32 = jnp.float32
    qp = q_ref[0]
    ptst = ptst_ref[0]

    qx = qp[:, 0:1]
    qy = qp[:, 1:2]
    qz = qp[:, 2:3]
    kx = ptst[0:1, :]
    ky = ptst[1:2, :]
    kz = ptst[2:3, :]

    q2 = qx * qx + qy * qy + qz * qz
    k2 = kx * kx + ky * ky + kz * kz
    dot = jax.lax.dot_general(
        qp.astype(jnp.bfloat16), ptst.astype(jnp.bfloat16),
        (((1,), (0,)), ((), ())),
        preferred_element_type=f32)
    d2 = (q2 + k2) - 2.0 * dot

    iota = jax.lax.broadcasted_iota(jnp.int32, (blk, n), 1)
    big = jnp.float32(jnp.inf)

    nbs = []
    for t in range(_K + 1):
        idx = jnp.argmin(d2, axis=1, keepdims=True)
        sel = iota == idx
        if t > 0:
            zero = jnp.float32(0.0)
            nbx = jnp.sum(jnp.where(sel, kx, zero), axis=1, keepdims=True)
            nby = jnp.sum(jnp.where(sel, ky, zero), axis=1, keepdims=True)
            nbz = jnp.sum(jnp.where(sel, kz, zero), axis=1, keepdims=True)
            nbs.append(jnp.concatenate([nbx, nby, nbz], axis=1))
        d2 = jnp.where(sel, big, d2)

    nb_flat = jnp.concatenate(nbs, axis=1)
    rels = [nb - qp for nb in nbs]
    rel_flat = jnp.concatenate(rels, axis=1)
    dists = jnp.concatenate(
        [jnp.sqrt(jnp.sum(r * r, axis=1, keepdims=True)) for r in rels],
        axis=1)

    mean = nbs[0]
    for nb in nbs[1:]:
        mean = mean + nb
    mean = mean * (1.0 / _K)
    c00 = c11 = c22 = c01 = c02 = c12 = jnp.zeros((blk, 1), f32)
    for nb in nbs:
        cx = nb[:, 0:1] - mean[:, 0:1]
        cy = nb[:, 1:2] - mean[:, 1:2]
        cz = nb[:, 2:3] - mean[:, 2:3]
        c00 = c00 + cx * cx
        c11 = c11 + cy * cy
        c22 = c22 + cz * cz
        c01 = c01 + cx * cy
        c02 = c02 + cx * cz
        c12 = c12 + cy * cz
    s = 1.0 / (_K - 1)
    l1, l2, l3 = _eigvals_sym3(c00 * s, c11 * s, c22 * s,
                               c01 * s, c02 * s, c12 * s)
    inv_l1 = 1.0 / l1
    linearity = (l1 - l2) * inv_l1
    planarity = (l2 - l3) * inv_l1
    scattering = l3 * inv_l1

    feats = jnp.concatenate(
        [qp, nb_flat, rel_flat, dists, linearity, planarity, scattering],
        axis=1)

    h = jax.lax.dot_general(
        feats.astype(jnp.bfloat16), w1t_ref[0].astype(jnp.bfloat16),
        (((1,), (0,)), ((), ())),
        preferred_element_type=f32) + b1_ref[0]
    h = jnp.maximum(h, 0.0)
    o = jax.lax.dot_general(
        h.astype(jnp.bfloat16), w2t_ref[0].astype(jnp.bfloat16),
        (((1,), (0,)), ((), ())),
        preferred_element_type=f32) + b2_ref[0]
    out_ref[0] = jnp.maximum(o, 0.0)


@jax.jit
def kernel(input_cloud, W1, b1, W2, b2):
    B, N, _ = input_cloud.shape
    BLK = 1024
    pts_t = jnp.transpose(input_cloud, (0, 2, 1))
    w1t = jnp.transpose(W1)[None]
    w2t = jnp.transpose(W2)[None]
    b1r = b1[None, None, :]
    b2r = b2[None, None, :]

    grid = (B, N // BLK)
    out = pl.pallas_call(
        functools.partial(_lge_kernel, blk=BLK, n=N),
        grid=grid,
        in_specs=[
            pl.BlockSpec((1, BLK, 3), lambda b, i: (b, i, 0)),
            pl.BlockSpec((1, 3, N), lambda b, i: (b, 0, 0)),
            pl.BlockSpec((1, 76, 64), lambda b, i: (0, 0, 0)),
            pl.BlockSpec((1, 1, 64), lambda b, i: (0, 0, 0)),
            pl.BlockSpec((1, 64, 3), lambda b, i: (0, 0, 0)),
            pl.BlockSpec((1, 1, 3), lambda b, i: (0, 0, 0)),
        ],
        out_specs=pl.BlockSpec((1, BLK, 3), lambda b, i: (b, i, 0)),
        out_shape=jax.ShapeDtypeStruct((B, N, 3), jnp.float32),
        compiler_params=pltpu.CompilerParams(
            dimension_semantics=("parallel", "parallel")),
    )(input_cloud, pts_t, w1t, b1r, w2t, b2r)
    return out


# speedup vs baseline: 49.2090x; 1.4607x over previous
import functools

import jax
import jax.numpy as jnp
from jax import lax
from jax.experimental import pallas as pl
from jax.experimental.pallas import tpu as pltpu
from jax.experimental.pallas import tpu_sc as plsc

_K = 10
_PAD = 128


def _knn_kernel(q_ref, ptst_ref, out_ref, *, blk, n):
    f32 = jnp.float32
    qp = q_ref[0]
    ptst = ptst_ref[0]

    qx = qp[:, 0:1]
    qy = qp[:, 1:2]
    qz = qp[:, 2:3]
    kx = ptst[0:1, :]
    ky = ptst[1:2, :]
    kz = ptst[2:3, :]

    q2 = qx * qx + qy * qy + qz * qz
    k2 = kx * kx + ky * ky + kz * kz
    dot = jax.lax.dot_general(
        qp.astype(jnp.bfloat16), ptst.astype(jnp.bfloat16),
        (((1,), (0,)), ((), ())),
        preferred_element_type=f32)
    d2 = (q2 + k2) - 2.0 * dot

    iota = jax.lax.broadcasted_iota(jnp.int32, (blk, n), 1)
    big = jnp.float32(jnp.inf)
    base = pl.program_id(0) * n

    idxs = []
    for t in range(_K + 1):
        idx = jnp.argmin(d2, axis=1, keepdims=True)
        if t > 0:
            idxs.append(idx + base)
        d2 = jnp.where(iota == idx, big, d2)
    out_ref[0] = jnp.concatenate(idxs, axis=1)


def _feat_kernel(q_ref, g_ref, w1t_ref, b1_ref, w2t_ref, b2_ref, out_ref):
    f32 = jnp.float32
    qp = q_ref[0]
    g = g_ref[0]

    nbs = [g[:, k * _PAD:k * _PAD + 3] for k in range(_K)]
    nb_flat = jnp.concatenate(nbs, axis=1)
    rels = [nb - qp for nb in nbs]
    rel_flat = jnp.concatenate(rels, axis=1)
    dists = jnp.concatenate(
        [jnp.sqrt(jnp.sum(r * r, axis=1, keepdims=True)) for r in rels],
        axis=1)

    mean = nbs[0]
    for nb in nbs[1:]:
        mean = mean + nb
    mean = mean * (1.0 / _K)
    blk = qp.shape[0]
    c00 = c11 = c22 = c01 = c02 = c12 = jnp.zeros((blk, 1), f32)
    for nb in nbs:
        cx = nb[:, 0:1] - mean[:, 0:1]
        cy = nb[:, 1:2] - mean[:, 1:2]
        cz = nb[:, 2:3] - mean[:, 2:3]
        c00 = c00 + cx * cx
        c11 = c11 + cy * cy
        c22 = c22 + cz * cz
        c01 = c01 + cx * cy
        c02 = c02 + cx * cz
        c12 = c12 + cy * cz
    s = 1.0 / (_K - 1)
    l1, l2, l3 = _eigvals_sym3(c00 * s, c11 * s, c22 * s,
                               c01 * s, c02 * s, c12 * s)
    inv_l1 = 1.0 / l1
    linearity = (l1 - l2) * inv_l1
    planarity = (l2 - l3) * inv_l1
    scattering = l3 * inv_l1

    feats = jnp.concatenate(
        [qp, nb_flat, rel_flat, dists, linearity, planarity, scattering],
        axis=1)

    h = jax.lax.dot_general(
        feats.astype(jnp.bfloat16), w1t_ref[0].astype(jnp.bfloat16),
        (((1,), (0,)), ((), ())),
        preferred_element_type=f32) + b1_ref[0]
    h = jnp.maximum(h, 0.0)
    o = jax.lax.dot_general(
        h.astype(jnp.bfloat16), w2t_ref[0].astype(jnp.bfloat16),
        (((1,), (0,)), ((), ())),
        preferred_element_type=f32) + b2_ref[0]
    out_ref[0] = jnp.maximum(o, 0.0)


def _eigvals_sym3(a, b, c, d, e, f):
    q = (a + b + c) * (1.0 / 3.0)
    p1 = d * d + e * e + f * f
    aq = a - q
    bq = b - q
    cq = c - q
    p2 = aq * aq + bq * bq + cq * cq + 2.0 * p1
    p = jnp.sqrt(p2 * (1.0 / 6.0))
    ps = jnp.maximum(p, 1e-30)
    inv = 1.0 / ps
    b00 = aq * inv
    b11 = bq * inv
    b22 = cq * inv
    b01 = d * inv
    b02 = e * inv
    b12 = f * inv
    detb = (
        b00 * (b11 * b22 - b12 * b12)
        - b01 * (b01 * b22 - b12 * b02)
        + b02 * (b01 * b12 - b11 * b02)
    )
    r = jnp.clip(0.5 * detb, -1.0, 1.0)
    phi = jnp.arctan2(jnp.sqrt(jnp.maximum(1.0 - r * r, 0.0)), r) * (1.0 / 3.0)
    two_pi_3 = 2.0943951023931953
    l1 = q + 2.0 * p * jnp.cos(phi)
    l3 = q + 2.0 * p * jnp.cos(phi + two_pi_3)
    l2 = 3.0 * q - l1 - l3
    return l1, l2, l3


def _sc_gather(table, idx_flat):
    info = plsc.get_sparse_core_info()
    nw = info.num_cores * info.num_subcores
    m = idx_flat.shape[0]
    b_per_w = m // nw
    mesh = plsc.VectorSubcoreMesh(core_axis_name="c", subcore_axis_name="s")

    ch = 512
    n_ch = b_per_w // ch

    @functools.partial(
        pl.kernel, mesh=mesh,
        out_type=jax.ShapeDtypeStruct((m, _PAD), jnp.float32),
        scratch_types=[
            pltpu.VMEM((ch,), jnp.int32),
            pltpu.VMEM((ch, _PAD), jnp.float32),
            pltpu.SemaphoreType.DMA,
        ],
    )
    def k(table_hbm, idx_hbm, out_hbm, idx_v, rows_v, sem):
        wid = lax.axis_index("s") * info.num_cores + lax.axis_index("c")
        base = wid * b_per_w
        for c in range(n_ch):
            off = base + c * ch
            pltpu.sync_copy(idx_hbm.at[pl.ds(off, ch)], idx_v)
            pltpu.async_copy(table_hbm.at[idx_v], rows_v, sem).wait()
            pltpu.sync_copy(rows_v, out_hbm.at[pl.ds(off, ch)])

    return k(table, idx_flat)


@jax.jit
def kernel(input_cloud, W1, b1, W2, b2):
    B, N, _ = input_cloud.shape
    BLK = 1024
    pts_t = jnp.transpose(input_cloud, (0, 2, 1))

    knn_idx = pl.pallas_call(
        functools.partial(_knn_kernel, blk=BLK, n=N),
        grid=(B, N // BLK),
        in_specs=[
            pl.BlockSpec((1, BLK, 3), lambda b, i: (b, i, 0)),
            pl.BlockSpec((1, 3, N), lambda b, i: (b, 0, 0)),
        ],
        out_specs=pl.BlockSpec((1, BLK, _K), lambda b, i: (b, i, 0)),
        out_shape=jax.ShapeDtypeStruct((B, N, _K), jnp.int32),
        compiler_params=pltpu.CompilerParams(
            dimension_semantics=("parallel", "parallel")),
    )(input_cloud, pts_t)

    table = jnp.pad(input_cloud.reshape(B * N, 3), ((0, 0), (0, _PAD - 3)))
    gathered = _sc_gather(table, knn_idx.reshape(B * N * _K))

    w1t = jnp.transpose(W1)[None]
    w2t = jnp.transpose(W2)[None]
    b1r = b1[None, None, :]
    b2r = b2[None, None, :]
    g3 = gathered.reshape(B, N, _K * _PAD)

    out = pl.pallas_call(
        _feat_kernel,
        grid=(B, N // BLK),
        in_specs=[
            pl.BlockSpec((1, BLK, 3), lambda b, i: (b, i, 0)),
            pl.BlockSpec((1, BLK, _K * _PAD), lambda b, i: (b, i, 0)),
            pl.BlockSpec((1, 76, 64), lambda b, i: (0, 0, 0)),
            pl.BlockSpec((1, 1, 64), lambda b, i: (0, 0, 0)),
            pl.BlockSpec((1, 64, 3), lambda b, i: (0, 0, 0)),
            pl.BlockSpec((1, 1, 3), lambda b, i: (0, 0, 0)),
        ],
        out_specs=pl.BlockSpec((1, BLK, 3), lambda b, i: (b, i, 0)),
        out_shape=jax.ShapeDtypeStruct((B, N, 3), jnp.float32),
        compiler_params=pltpu.CompilerParams(
            dimension_semantics=("parallel", "parallel")),
    )(input_cloud, g3, w1t, b1r, w2t, b2r)
    return out
